# baseline (speedup 1.0000x reference)
import functools

import jax
import jax.numpy as jnp
from jax import lax
from jax.experimental import pallas as pl
from jax.experimental.pallas import tpu as pltpu
from jax.experimental.pallas import tpu_sc as plsc

_NC = 2
_NS = 16
_NW = _NC * _NS
_CHUNKS = 25


def _tc_copy_body(x_ref, ox_ref):
    ox_ref[...] = x_ref[...]


def _tc_copy(x):
    grid = 5
    bx = x.shape[0] // grid
    return pl.pallas_call(
        _tc_copy_body,
        grid=(grid,),
        in_specs=[pl.BlockSpec((bx, x.shape[1]), lambda i: (i, 0))],
        out_specs=pl.BlockSpec((bx, x.shape[1]), lambda i: (i, 0)),
        out_shape=jax.ShapeDtypeStruct(x.shape, x.dtype),
    )(x)


def _sc_copy_body(rows, chunk, e_hbm, oe_hbm, ebuf, si0, si1, so0, so1):
    wid = lax.axis_index("s") * _NC + lax.axis_index("c")
    base = wid * rows
    s_in = (si0, si1)
    s_out = (so0, so1)

    def rd(c, slot):
        return pltpu.make_async_copy(
            e_hbm.at[pl.ds(base + c * chunk, chunk), :], ebuf.at[slot],
            s_in[slot])

    def wr(c, slot):
        return pltpu.make_async_copy(
            ebuf.at[slot], oe_hbm.at[pl.ds(base + c * chunk, chunk), :],
            s_out[slot])

    rd(0, 0).start()
    for c in range(_CHUNKS):
        slot = c % 2
        rd(c, slot).wait()
        wr(c, slot).start()
        if c + 1 < _CHUNKS:
            if c >= 1:
                wr(c - 1, 1 - slot).wait()
            rd(c + 1, 1 - slot).start()
    wr(_CHUNKS - 2, _CHUNKS % 2).wait()
    wr(_CHUNKS - 1, (_CHUNKS - 1) % 2).wait()


def _sc_copy(edge_attr):
    n_edges, d_edge = edge_attr.shape
    rows = n_edges // _NW
    chunk = rows // _CHUNKS
    mesh = plsc.VectorSubcoreMesh(core_axis_name="c", subcore_axis_name="s")
    body = functools.partial(_sc_copy_body, rows, chunk)
    return pl.kernel(
        body,
        out_type=jax.ShapeDtypeStruct(edge_attr.shape, edge_attr.dtype),
        mesh=mesh,
        scratch_types=[
            pltpu.VMEM((2, chunk, d_edge), edge_attr.dtype),
            pltpu.SemaphoreType.DMA,
            pltpu.SemaphoreType.DMA,
            pltpu.SemaphoreType.DMA,
            pltpu.SemaphoreType.DMA,
        ],
    )(edge_attr)


def kernel(x, edge_index, edge_attr):
    del edge_index
    return (_tc_copy(x), _sc_copy(edge_attr))
